# baseline (speedup 1.0000x reference)
import functools

import jax
import jax.numpy as jnp
import numpy as np
from jax.experimental import pallas as pl
from jax.experimental.pallas import tpu as pltpu

_ACTION_DIM = 6
_G = 64
_LANE = 128


def _pack_matrices():
    H = np.zeros((2, 3, 12, _G, _LANE), np.float32)
    for o in range(2):
        for t in range(3):
            for l in range(_LANE):
                f = _LANE * t + l
                H[o, t, 6 * o + f % 6, f // 6, l] = 1.0
    return H.reshape(2 * 3 * 12 * _G, _LANE)


_PACK = _pack_matrices()


def _actor_kernel(x_ref, w1_ref, b1_ref, w2_ref, b2_ref, w3_ref, b3_ref,
                  h_ref, mean_ref, std_ref, *, action_dim):
    x = x_ref[...]
    h1 = jnp.maximum(
        jnp.dot(x, w1_ref[...], preferred_element_type=jnp.float32)
        + b1_ref[...], 0.0)
    h2 = jnp.maximum(
        jnp.dot(h1, w2_ref[...], preferred_element_type=jnp.float32)
        + b2_ref[...], 0.0)
    raw = jnp.dot(h2, w3_ref[...],
                  preferred_element_type=jnp.float32) + b3_ref[...]

    TB = raw.shape[0]
    S = TB // _G
    raw3 = jnp.reshape(raw, (S, _G, 2 * action_dim))
    raw3t = jnp.transpose(raw3, (0, 2, 1))
    sl = [jnp.reshape(raw3t[:, j, :], (S, _G))
          for j in range(2 * action_dim)]

    def packed(o):
        rows = []
        for t in range(3):
            acc = None
            for j6 in range(action_dim):
                j = action_dim * o + j6
                base = ((o * 3 + t) * 12 + j) * _G
                hj = h_ref[pl.ds(base, _G), :]
                term = jnp.dot(sl[j], hj,
                               preferred_element_type=jnp.float32)
                acc = term if acc is None else acc + term
            rows.append(acc)
        st = jnp.stack(rows, axis=1)
        return jnp.reshape(st, (3 * S, _LANE))

    p_mean = packed(0)
    p_std = packed(1)
    mean_ref[...] = jnp.clip(p_mean, -100.0, 100.0)
    std_ref[...] = jnp.clip(
        jnp.exp(jnp.clip(p_std, -20.0, 2.0)), 0.01, 100.0)


def _pick_tile(batch):
    for tb in (8192, 4096, 2048, 1024, 512, 256, 128, 64):
        if batch % tb == 0 and batch // tb >= 2 and tb % _G == 0:
            return tb
    return batch


def kernel(state, w1, b1, w2, b2, w3t, b3t):
    B, K = state.shape
    H = w1.shape[1]
    A = _ACTION_DIM

    w3b = jnp.transpose(w3t[:2 * A, :])
    b3 = jnp.transpose(b3t[:2 * A, :])
    hmat = jnp.asarray(_PACK)

    TB = _pick_tile(B)
    n_tiles = B // TB
    PR = TB * A // _LANE

    def resident(arr):
        return pl.BlockSpec(arr.shape, lambda i: (0,) * arr.ndim)

    in_specs = [
        pl.BlockSpec((TB, K), lambda i: (i, 0)),
        resident(w1), resident(b1),
        resident(w2), resident(b2),
        resident(w3b), resident(b3),
        resident(hmat),
    ]
    out_specs = [
        pl.BlockSpec((PR, _LANE), lambda i: (i, 0)),
        pl.BlockSpec((PR, _LANE), lambda i: (i, 0)),
    ]

    param_bytes = sum(int(np.prod(p.shape)) * p.dtype.itemsize
                      for p in (w1, b1, w2, b2, w3b, b3, hmat))
    cost = pl.CostEstimate(
        flops=2 * B * (K * H + H * H + H * 2 * A + 12 * _G * A),
        transcendentals=B * A,
        bytes_accessed=4 * (B * K + 2 * B * A) + param_bytes,
    )

    mean_p, std_p = pl.pallas_call(
        functools.partial(_actor_kernel, action_dim=A),
        out_shape=[jax.ShapeDtypeStruct((B * A // _LANE, _LANE), jnp.float32),
                   jax.ShapeDtypeStruct((B * A // _LANE, _LANE), jnp.float32)],
        grid=(n_tiles,),
        in_specs=in_specs,
        out_specs=out_specs,
        compiler_params=pltpu.CompilerParams(
            dimension_semantics=("parallel",)),
        cost_estimate=cost,
    )(state, w1, b1, w2, b2, w3b, b3, hmat)
    return jnp.reshape(mean_p, (B, A)), jnp.reshape(std_p, (B, A))
